# baseline (speedup 1.0000x reference)
import functools

import jax
import jax.numpy as jnp
from jax import lax
from jax.experimental import pallas as pl
from jax.experimental.pallas import tpu as pltpu

_HB = 64


def _dn_kernel(x_ref, xbot_ref, w_ref, b_ref, tw_ref, inv_ref, o_ref,
               prev_ref, *, Hb, W, nj):
    x = x_ref[0]
    C = x.shape[0]
    j = pl.program_id(1)

    top = jnp.where(j > 0, prev_ref[...], 0.0)[:, None, :]
    bot = jnp.where(j < nj - 1, xbot_ref[0, :, 0:1, :], 0.0)
    v = x + jnp.concatenate([top, x[:, :-1, :]], axis=1) \
          + jnp.concatenate([x[:, 1:, :], bot], axis=1)
    prev_ref[...] = x[:, Hb - 1, :]

    t2 = lax.dot_general(w_ref[...], v.reshape(C, Hb * W),
                         (((1,), (0,)), ((), ())),
                         preferred_element_type=jnp.float32)
    t3 = t2.reshape(C, Hb, W)

    s = lax.dot_general(t3.reshape(C * Hb, W), tw_ref[...],
                        (((1,), (0,)), ((), ())),
                        preferred_element_type=jnp.float32).reshape(C, Hb, W)

    o_ref[0] = x + s * inv_ref[...] + b_ref[...]


def kernel(x, conv_w, conv_b):
    B, C, H, W = x.shape
    f32 = jnp.float32
    Hb = _HB
    nj = H // Hb
    nt8 = H // 8

    i = jnp.arange(W)
    tw = (jnp.abs(i[:, None] - i[None, :]) <= 1).astype(f32)
    ch = jnp.where((jnp.arange(H) == 0) | (jnp.arange(H) == H - 1), 2.0, 3.0)
    cw = jnp.where((i == 0) | (i == W - 1), 2.0, 3.0)
    inv = (1.0 / (ch[:, None] * cw[None, :])).astype(f32)[None]
    b3 = conv_b.reshape(C, 1, 1)

    return pl.pallas_call(
        functools.partial(_dn_kernel, Hb=Hb, W=W, nj=nj),
        grid=(B, nj),
        in_specs=[
            pl.BlockSpec((1, C, Hb, W), lambda b, j: (b, 0, j, 0)),
            pl.BlockSpec((1, C, 8, W),
                         lambda b, j: (b, 0,
                                       jnp.minimum((j + 1) * (Hb // 8),
                                                   nt8 - 1), 0)),
            pl.BlockSpec((C, C), lambda b, j: (0, 0)),
            pl.BlockSpec((C, 1, 1), lambda b, j: (0, 0, 0)),
            pl.BlockSpec((W, W), lambda b, j: (0, 0)),
            pl.BlockSpec((1, Hb, W), lambda b, j: (0, j, 0)),
        ],
        out_specs=pl.BlockSpec((1, C, Hb, W), lambda b, j: (b, 0, j, 0)),
        out_shape=jax.ShapeDtypeStruct((B, C, H, W), x.dtype),
        scratch_shapes=[pltpu.VMEM((C, W), f32)],
        compiler_params=pltpu.CompilerParams(
            dimension_semantics=("parallel", "arbitrary"),
        ),
    )(x, x, conv_w, b3, tw, inv)


# speedup vs baseline: 1.2598x; 1.2598x over previous
import functools

import jax
import jax.numpy as jnp
from jax import lax
from jax.experimental import pallas as pl
from jax.experimental.pallas import tpu as pltpu


def _dn_kernel(x_ref, w_ref, b_ref, tw_ref, inv_ref, o_ref, *, H, W):
    x = x_ref[0]
    C = x.shape[0]

    zh = jnp.zeros((C, 1, W), x.dtype)
    v = x + jnp.concatenate([zh, x[:, :-1, :]], axis=1) \
          + jnp.concatenate([x[:, 1:, :], zh], axis=1)

    v2 = v.astype(jnp.bfloat16).reshape(C, H * W)
    t2 = lax.dot_general(w_ref[...], v2, (((1,), (0,)), ((), ())),
                         preferred_element_type=jnp.float32)
    t3 = t2.astype(jnp.bfloat16).reshape(C, H, W)

    s = lax.dot_general(t3.reshape(C * H, W), tw_ref[...],
                        (((1,), (0,)), ((), ())),
                        preferred_element_type=jnp.float32).reshape(C, H, W)

    o_ref[0] = x + s * inv_ref[...] + b_ref[...]


def kernel(x, conv_w, conv_b):
    B, C, H, W = x.shape
    f32 = jnp.float32

    i = jnp.arange(W)
    tw = (jnp.abs(i[:, None] - i[None, :]) <= 1).astype(jnp.bfloat16)
    ch = jnp.where((jnp.arange(H) == 0) | (jnp.arange(H) == H - 1), 2.0, 3.0)
    cw = jnp.where((i == 0) | (i == W - 1), 2.0, 3.0)
    inv = (1.0 / (ch[:, None] * cw[None, :])).astype(f32)[None]
    b3 = conv_b.reshape(C, 1, 1)
    w16 = conv_w.astype(jnp.bfloat16)

    return pl.pallas_call(
        functools.partial(_dn_kernel, H=H, W=W),
        grid=(B,),
        in_specs=[
            pl.BlockSpec((1, C, H, W), lambda b: (b, 0, 0, 0)),
            pl.BlockSpec((C, C), lambda b: (0, 0)),
            pl.BlockSpec((C, 1, 1), lambda b: (0, 0, 0)),
            pl.BlockSpec((W, W), lambda b: (0, 0)),
            pl.BlockSpec((1, H, W), lambda b: (0, 0, 0)),
        ],
        out_specs=pl.BlockSpec((1, C, H, W), lambda b: (b, 0, 0, 0)),
        out_shape=jax.ShapeDtypeStruct((B, C, H, W), x.dtype),
        compiler_params=pltpu.CompilerParams(
            dimension_semantics=("parallel",),
        ),
    )(x, w16, b3, tw, inv)


# speedup vs baseline: 1.3271x; 1.0534x over previous
import functools

import jax
import jax.numpy as jnp
from jax import lax
from jax.experimental import pallas as pl
from jax.experimental.pallas import tpu as pltpu


def _dn_kernel(x_ref, w_ref, b_ref, tw_ref, inv_ref, o_ref, *, H, W):
    x = x_ref[0]
    C = x.shape[0]

    x16 = x.astype(jnp.bfloat16)
    zh = jnp.zeros((C, 1, W), jnp.bfloat16)
    v = x16 + jnp.concatenate([zh, x16[:, :-1, :]], axis=1) \
            + jnp.concatenate([x16[:, 1:, :], zh], axis=1)

    v2 = v.reshape(C, H * W)
    t2 = lax.dot_general(w_ref[...], v2, (((1,), (0,)), ((), ())),
                         preferred_element_type=jnp.float32)
    t3 = t2.astype(jnp.bfloat16).reshape(C, H, W)

    s = lax.dot_general(t3.reshape(C * H, W), tw_ref[...],
                        (((1,), (0,)), ((), ())),
                        preferred_element_type=jnp.float32).reshape(C, H, W)

    o_ref[0] = x + s * inv_ref[...] + b_ref[...]


def kernel(x, conv_w, conv_b):
    B, C, H, W = x.shape
    f32 = jnp.float32

    i = jnp.arange(W)
    tw = (jnp.abs(i[:, None] - i[None, :]) <= 1).astype(jnp.bfloat16)
    ch = jnp.where((jnp.arange(H) == 0) | (jnp.arange(H) == H - 1), 2.0, 3.0)
    cw = jnp.where((i == 0) | (i == W - 1), 2.0, 3.0)
    inv = (1.0 / (ch[:, None] * cw[None, :])).astype(f32)[None]
    b3 = conv_b.reshape(C, 1, 1)
    w16 = conv_w.astype(jnp.bfloat16)

    return pl.pallas_call(
        functools.partial(_dn_kernel, H=H, W=W),
        grid=(B,),
        in_specs=[
            pl.BlockSpec((1, C, H, W), lambda b: (b, 0, 0, 0)),
            pl.BlockSpec((C, C), lambda b: (0, 0)),
            pl.BlockSpec((C, 1, 1), lambda b: (0, 0, 0)),
            pl.BlockSpec((W, W), lambda b: (0, 0)),
            pl.BlockSpec((1, H, W), lambda b: (0, 0, 0)),
        ],
        out_specs=pl.BlockSpec((1, C, H, W), lambda b: (b, 0, 0, 0)),
        out_shape=jax.ShapeDtypeStruct((B, C, H, W), x.dtype),
        compiler_params=pltpu.CompilerParams(
            dimension_semantics=("parallel",),
        ),
    )(x, w16, b3, tw, inv)
